# baseline (speedup 1.0000x reference)
import numpy as np

import jax
import jax.numpy as jnp
from jax import lax
from jax.experimental import pallas as pl
from jax.experimental.pallas import tpu as pltpu

_B, _Q, _C = 2, 100, 81
_N = 20
_h = _w = 128
_H = _W = 512
_NQCH = 4
_CH = _H // _NQCH
_KW = 64
_WSTART = tuple(min(max(32 * q - 8, 0), _h - _KW) for q in range(_NQCH))


def _interp_matrix(out_size: int, in_size: int) -> np.ndarray:
    o = np.arange(out_size, dtype=np.float32)
    src = (o + 0.5) * (in_size / out_size) - 0.5
    i0f = np.floor(src)
    frac = (src - i0f).astype(np.float32)
    i0 = np.clip(i0f.astype(np.int64), 0, in_size - 1)
    i1 = np.clip(i0f.astype(np.int64) + 1, 0, in_size - 1)
    A = np.zeros((out_size, in_size), dtype=np.float32)
    A[o.astype(np.int64), i0] += 1.0 - frac
    A[o.astype(np.int64), i1] += frac
    return A


_A_NP = _interp_matrix(_H, _h)
_ABAND_NP = np.stack(
    [_A_NP[q * _CH:(q + 1) * _CH, _WSTART[q]:_WSTART[q] + _KW]
     for q in range(_NQCH)], axis=0)
_ABAND_NP = _ABAND_NP.reshape(_NQCH * _CH, _KW)


def _match_body(labels_ref, logits_ref, idx_ref, ss_ref):
    logits = logits_ref[0]
    mx = jnp.max(logits, axis=-1, keepdims=True)
    e = jnp.exp(logits - mx)
    prob = e / jnp.sum(e, axis=-1, keepdims=True)
    labels = labels_ref[0]
    iota_c = lax.broadcasted_iota(jnp.int32, (_C, _N), 0)
    onehot = (iota_c == labels).astype(jnp.float32)
    probT = lax.dot_general(onehot, prob, (((0,), (1,)), ((), ())),
                            precision=lax.Precision.HIGHEST,
                            preferred_element_type=jnp.float32)

    iota_row = lax.broadcasted_iota(jnp.int32, (_N, _Q), 0)
    iota_lane = lax.broadcasted_iota(jnp.int32, (1, _Q), 1)
    iota_tn = lax.broadcasted_iota(jnp.int32, (1, _N), 1)

    def step(t, carry):
        used, idxv, ssv = carry
        row = jnp.sum(jnp.where(iota_row == t, probT, 0.0), axis=0,
                      keepdims=True)
        c = jnp.where(used > 0.5, -jnp.inf, row)
        m = jnp.max(c)
        j = jnp.min(jnp.where(c == m, iota_lane, _Q))
        sel = iota_tn == t
        idxv = jnp.where(sel, j, idxv)
        ssv = jnp.where(sel, m, ssv)
        used = jnp.where(iota_lane == j, 1.0, used)
        return used, idxv, ssv

    used0 = jnp.zeros((1, _Q), dtype=jnp.float32)
    _, idxv, ssv = lax.fori_loop(
        0, _N, step,
        (used0, jnp.zeros((1, _N), jnp.int32), jnp.zeros((1, _N), jnp.float32)))
    idx_ref[0] = idxv
    ss_ref[0] = ssv


def _split3(x):
    hi = x.astype(jnp.bfloat16)
    r1 = x - hi.astype(jnp.float32)
    mid = r1.astype(jnp.bfloat16)
    lo = (r1 - mid.astype(jnp.float32)).astype(jnp.bfloat16)
    return hi, mid, lo


def _split3_dot(x, a_bf16):
    hi, mid, lo = _split3(x)
    d = lambda u: jnp.dot(u, a_bf16, preferred_element_type=jnp.float32)
    return d(hi) + d(mid) + d(lo)


def _split3_dot_rhs(a_bf16, x):
    hi, mid, lo = _split3(x)
    d = lambda u: jnp.dot(a_bf16, u, preferred_element_type=jnp.float32)
    return d(hi) + d(mid) + d(lo)


def _dense_body(idx_s, ss_s, pred_ref, tgt_ref, At_ref, Ab_ref, out_ref,
                score_ref):
    b = pl.program_id(0)
    n = pl.program_id(1)
    x = pred_ref[0, 0]
    soft = 1.0 / (1.0 + jnp.exp(-x))
    hard = (soft > 0.5).astype(jnp.float32)
    num = jnp.sum(soft * hard)
    den = jnp.sum(hard)
    mask_score = num / (den + 1e-6)
    score = ss_s[b * _N + n] * mask_score

    At = At_ref[...]
    wide = _split3_dot(soft, At)
    for q in range(_NQCH):
        aq = Ab_ref[q * _CH:(q + 1) * _CH, :]
        wq = wide[_WSTART[q]:_WSTART[q] + _KW, :]
        upq = _split3_dot_rhs(aq, wq)
        out_ref[0, 0, q * _CH:(q + 1) * _CH, :] = jnp.where(
            upq > 0.5, tgt_ref[0, 0, q * _CH:(q + 1) * _CH, :], 0.0)
    score_ref[...] = jnp.full((1, 1, 1, 128), score, dtype=jnp.float32)


def kernel(pred_logits, pred_masks, tgt_masks, tgt_labels):
    B, Q, C = pred_logits.shape
    N = tgt_masks.shape[1]
    labels3 = tgt_labels.astype(jnp.int32).reshape(B, 1, N)

    idx, ss = pl.pallas_call(
        _match_body,
        grid=(B,),
        in_specs=[
            pl.BlockSpec((1, 1, N), lambda b: (b, 0, 0)),
            pl.BlockSpec((1, Q, C), lambda b: (b, 0, 0)),
        ],
        out_specs=[
            pl.BlockSpec((1, 1, N), lambda b: (b, 0, 0)),
            pl.BlockSpec((1, 1, N), lambda b: (b, 0, 0)),
        ],
        out_shape=[
            jax.ShapeDtypeStruct((B, 1, N), jnp.int32),
            jax.ShapeDtypeStruct((B, 1, N), jnp.float32),
        ],
    )(labels3, pred_logits)

    idx_flat = idx.reshape(B * N)
    ss_flat = ss.reshape(B * N)
    At = jnp.asarray(_A_NP.T.copy()).astype(jnp.bfloat16)
    Ab = jnp.asarray(_ABAND_NP).astype(jnp.bfloat16)

    masks, scores_pad = pl.pallas_call(
        _dense_body,
        grid_spec=pltpu.PrefetchScalarGridSpec(
            num_scalar_prefetch=2,
            grid=(B, N),
            in_specs=[
                pl.BlockSpec((1, 1, _h, _w),
                             lambda b, n, idx_s, ss_s: (b, idx_s[b * N + n], 0, 0)),
                pl.BlockSpec((1, 1, _H, _W),
                             lambda b, n, idx_s, ss_s: (b, n, 0, 0)),
                pl.BlockSpec((_h, _W), lambda b, n, idx_s, ss_s: (0, 0)),
                pl.BlockSpec((_H, _KW), lambda b, n, idx_s, ss_s: (0, 0)),
            ],
            out_specs=[
                pl.BlockSpec((1, 1, _H, _W),
                             lambda b, n, idx_s, ss_s: (b, n, 0, 0)),
                pl.BlockSpec((1, 1, 1, 128),
                             lambda b, n, idx_s, ss_s: (b, n, 0, 0)),
            ],
        ),
        out_shape=[
            jax.ShapeDtypeStruct((B, N, _H, _W), jnp.float32),
            jax.ShapeDtypeStruct((B, N, 1, 128), jnp.float32),
        ],
    )(idx_flat, ss_flat, pred_masks, tgt_masks, At, Ab)

    return scores_pad[:, :, 0, 0], masks


# speedup vs baseline: 1.2931x; 1.2931x over previous
import numpy as np

import jax
import jax.numpy as jnp
from jax import lax
from jax.experimental import pallas as pl
from jax.experimental.pallas import tpu as pltpu

_B, _Q, _C = 2, 100, 81
_N = 20
_h = _w = 128
_H = _W = 512
_NQCH = 4
_CH = _H // _NQCH
_KW = 64
_WSTART = tuple(min(max(32 * q - 8, 0), _h - _KW) for q in range(_NQCH))


def _interp_matrix(out_size: int, in_size: int) -> np.ndarray:
    o = np.arange(out_size, dtype=np.float32)
    src = (o + 0.5) * (in_size / out_size) - 0.5
    i0f = np.floor(src)
    frac = (src - i0f).astype(np.float32)
    i0 = np.clip(i0f.astype(np.int64), 0, in_size - 1)
    i1 = np.clip(i0f.astype(np.int64) + 1, 0, in_size - 1)
    A = np.zeros((out_size, in_size), dtype=np.float32)
    A[o.astype(np.int64), i0] += 1.0 - frac
    A[o.astype(np.int64), i1] += frac
    return A


_A_NP = _interp_matrix(_H, _h)
_ABAND_NP = np.stack(
    [_A_NP[q * _CH:(q + 1) * _CH, _WSTART[q]:_WSTART[q] + _KW]
     for q in range(_NQCH)], axis=0)
_ABAND_NP = _ABAND_NP.reshape(_NQCH * _CH, _KW)


def _match_body(labels_ref, logits_ref, idx_ref, ss_ref):
    logits = logits_ref[0]
    mx = jnp.max(logits, axis=-1, keepdims=True)
    e = jnp.exp(logits - mx)
    prob = e / jnp.sum(e, axis=-1, keepdims=True)
    labels = labels_ref[0]
    iota_c = lax.broadcasted_iota(jnp.int32, (_C, _N), 0)
    onehot = (iota_c == labels).astype(jnp.float32)
    probT = lax.dot_general(onehot, prob, (((0,), (1,)), ((), ())),
                            precision=lax.Precision.HIGHEST,
                            preferred_element_type=jnp.float32)

    iota_row = lax.broadcasted_iota(jnp.int32, (_N, _Q), 0)
    iota_lane = lax.broadcasted_iota(jnp.int32, (1, _Q), 1)
    iota_tn = lax.broadcasted_iota(jnp.int32, (1, _N), 1)

    def step(t, carry):
        used, idxv, ssv = carry
        row = jnp.sum(jnp.where(iota_row == t, probT, 0.0), axis=0,
                      keepdims=True)
        c = jnp.where(used > 0.5, -jnp.inf, row)
        m = jnp.max(c)
        j = jnp.min(jnp.where(c == m, iota_lane, _Q))
        sel = iota_tn == t
        idxv = jnp.where(sel, j, idxv)
        ssv = jnp.where(sel, m, ssv)
        used = jnp.where(iota_lane == j, 1.0, used)
        return used, idxv, ssv

    used0 = jnp.zeros((1, _Q), dtype=jnp.float32)
    _, idxv, ssv = lax.fori_loop(
        0, _N, step,
        (used0, jnp.zeros((1, _N), jnp.int32), jnp.zeros((1, _N), jnp.float32)))
    idx_ref[0] = idxv
    ss_ref[0] = ssv


def _split3(x):
    hi = x.astype(jnp.bfloat16)
    r1 = x - hi.astype(jnp.float32)
    mid = r1.astype(jnp.bfloat16)
    lo = (r1 - mid.astype(jnp.float32)).astype(jnp.bfloat16)
    return hi, mid, lo


def _split3_dot(x, a_bf16):
    hi, mid, lo = _split3(x)
    d = lambda u: jnp.dot(u, a_bf16, preferred_element_type=jnp.float32)
    return d(hi) + d(mid) + d(lo)


def _split3_dot_rhs(a_bf16, x):
    hi, mid, lo = _split3(x)
    d = lambda u: jnp.dot(a_bf16, u, preferred_element_type=jnp.float32)
    return d(hi) + d(mid) + d(lo)


_G = 4


def _dense_body(idx_s, ss_s, p0_ref, p1_ref, p2_ref, p3_ref, tgt_ref,
                At_ref, Ab_ref, out_ref, score_ref):
    b = pl.program_id(0)
    g = pl.program_id(1)
    At = At_ref[...]
    preds = (p0_ref, p1_ref, p2_ref, p3_ref)
    for j in range(_G):
        x = preds[j][0, 0]
        soft = 1.0 / (1.0 + jnp.exp(-x))
        hard = (soft > 0.5).astype(jnp.float32)
        num = jnp.sum(soft * hard)
        den = jnp.sum(hard)
        mask_score = num / (den + 1e-6)
        score = ss_s[b * _N + g * _G + j] * mask_score

        wide = _split3_dot(soft, At)
        for q in range(_NQCH):
            aq = Ab_ref[q * _CH:(q + 1) * _CH, :]
            wq = wide[_WSTART[q]:_WSTART[q] + _KW, :]
            upq = _split3_dot_rhs(aq, wq)
            out_ref[0, j, q * _CH:(q + 1) * _CH, :] = jnp.where(
                upq > 0.5, tgt_ref[0, j, q * _CH:(q + 1) * _CH, :], 0.0)
        score_ref[0, j] = jnp.full((1, 128), score, dtype=jnp.float32)


def kernel(pred_logits, pred_masks, tgt_masks, tgt_labels):
    B, Q, C = pred_logits.shape
    N = tgt_masks.shape[1]
    labels3 = tgt_labels.astype(jnp.int32).reshape(B, 1, N)

    idx, ss = pl.pallas_call(
        _match_body,
        grid=(B,),
        in_specs=[
            pl.BlockSpec((1, 1, N), lambda b: (b, 0, 0)),
            pl.BlockSpec((1, Q, C), lambda b: (b, 0, 0)),
        ],
        out_specs=[
            pl.BlockSpec((1, 1, N), lambda b: (b, 0, 0)),
            pl.BlockSpec((1, 1, N), lambda b: (b, 0, 0)),
        ],
        out_shape=[
            jax.ShapeDtypeStruct((B, 1, N), jnp.int32),
            jax.ShapeDtypeStruct((B, 1, N), jnp.float32),
        ],
    )(labels3, pred_logits)

    idx_flat = idx.reshape(B * N)
    ss_flat = ss.reshape(B * N)
    At = jnp.asarray(_A_NP.T.copy()).astype(jnp.bfloat16)
    Ab = jnp.asarray(_ABAND_NP).astype(jnp.bfloat16)

    def pred_spec(j):
        return pl.BlockSpec(
            (1, 1, _h, _w),
            lambda b, g, idx_s, ss_s: (b, idx_s[b * N + g * _G + j], 0, 0))

    masks, scores_pad = pl.pallas_call(
        _dense_body,
        grid_spec=pltpu.PrefetchScalarGridSpec(
            num_scalar_prefetch=2,
            grid=(B, N // _G),
            in_specs=[
                pred_spec(0), pred_spec(1), pred_spec(2), pred_spec(3),
                pl.BlockSpec((1, _G, _H, _W),
                             lambda b, g, idx_s, ss_s: (b, g, 0, 0)),
                pl.BlockSpec((_h, _W), lambda b, g, idx_s, ss_s: (0, 0)),
                pl.BlockSpec((_H, _KW), lambda b, g, idx_s, ss_s: (0, 0)),
            ],
            out_specs=[
                pl.BlockSpec((1, _G, _H, _W),
                             lambda b, g, idx_s, ss_s: (b, g, 0, 0)),
                pl.BlockSpec((1, _G, 1, 128),
                             lambda b, g, idx_s, ss_s: (b, g, 0, 0)),
            ],
        ),
        out_shape=[
            jax.ShapeDtypeStruct((B, N, _H, _W), jnp.float32),
            jax.ShapeDtypeStruct((B, N, 1, 128), jnp.float32),
        ],
    )(idx_flat, ss_flat, pred_masks, pred_masks, pred_masks, pred_masks,
      tgt_masks, At, Ab)

    return scores_pad[:, :, 0, 0], masks


# speedup vs baseline: 1.3058x; 1.0098x over previous
import numpy as np

import jax
import jax.numpy as jnp
from jax import lax
from jax.experimental import pallas as pl
from jax.experimental.pallas import tpu as pltpu

_B, _Q, _C = 2, 100, 81
_N = 20
_h = _w = 128
_H = _W = 512
_NQCH = 4
_CH = _H // _NQCH
_KW = 64
_WSTART = tuple(min(max(32 * q - 8, 0), _h - _KW) for q in range(_NQCH))


def _interp_matrix(out_size: int, in_size: int) -> np.ndarray:
    o = np.arange(out_size, dtype=np.float32)
    src = (o + 0.5) * (in_size / out_size) - 0.5
    i0f = np.floor(src)
    frac = (src - i0f).astype(np.float32)
    i0 = np.clip(i0f.astype(np.int64), 0, in_size - 1)
    i1 = np.clip(i0f.astype(np.int64) + 1, 0, in_size - 1)
    A = np.zeros((out_size, in_size), dtype=np.float32)
    A[o.astype(np.int64), i0] += 1.0 - frac
    A[o.astype(np.int64), i1] += frac
    return A


_A_NP = _interp_matrix(_H, _h)
_ABAND_NP = np.stack(
    [_A_NP[q * _CH:(q + 1) * _CH, _WSTART[q]:_WSTART[q] + _KW]
     for q in range(_NQCH)], axis=0)
_ABAND_NP = _ABAND_NP.reshape(_NQCH * _CH, _KW)


def _match_body(labels_ref, logits_ref, idx_ref, ss_ref):
    logits = logits_ref[0]
    mx = jnp.max(logits, axis=-1, keepdims=True)
    e = jnp.exp(logits - mx)
    prob = e / jnp.sum(e, axis=-1, keepdims=True)
    labels = labels_ref[0]
    iota_c = lax.broadcasted_iota(jnp.int32, (_C, _N), 0)
    onehot = (iota_c == labels).astype(jnp.float32)
    probT = lax.dot_general(onehot, prob, (((0,), (1,)), ((), ())),
                            precision=lax.Precision.HIGHEST,
                            preferred_element_type=jnp.float32)

    iota_row = lax.broadcasted_iota(jnp.int32, (_N, _Q), 0)
    iota_lane = lax.broadcasted_iota(jnp.int32, (1, _Q), 1)
    iota_tn = lax.broadcasted_iota(jnp.int32, (1, _N), 1)

    def step(t, carry):
        used, idxv, ssv = carry
        row = jnp.sum(jnp.where(iota_row == t, probT, 0.0), axis=0,
                      keepdims=True)
        c = jnp.where(used > 0.5, -jnp.inf, row)
        m = jnp.max(c)
        j = jnp.min(jnp.where(c == m, iota_lane, _Q))
        sel = iota_tn == t
        idxv = jnp.where(sel, j, idxv)
        ssv = jnp.where(sel, m, ssv)
        used = jnp.where(iota_lane == j, 1.0, used)
        return used, idxv, ssv

    used0 = jnp.zeros((1, _Q), dtype=jnp.float32)
    _, idxv, ssv = lax.fori_loop(
        0, _N, step,
        (used0, jnp.zeros((1, _N), jnp.int32), jnp.zeros((1, _N), jnp.float32)))
    idx_ref[0] = idxv
    ss_ref[0] = ssv


def _split3(x):
    hi = x.astype(jnp.bfloat16)
    r1 = x - hi.astype(jnp.float32)
    mid = r1.astype(jnp.bfloat16)
    lo = (r1 - mid.astype(jnp.float32)).astype(jnp.bfloat16)
    return hi, mid, lo


def _split3_dot(x, a_bf16):
    hi, mid, lo = _split3(x)
    d = lambda u: jnp.dot(u, a_bf16, preferred_element_type=jnp.float32)
    return d(hi) + d(mid) + d(lo)


def _split3_dot_rhs(a_bf16, x):
    hi, mid, lo = _split3(x)
    d = lambda u: jnp.dot(a_bf16, u, preferred_element_type=jnp.float32)
    return d(hi) + d(mid) + d(lo)


_G = 5


def _dense_body(idx_s, ss_s, p0_ref, p1_ref, p2_ref, p3_ref, p4_ref,
                tgt_ref, At_ref, Ab_ref, out_ref, score_ref):
    b = pl.program_id(0)
    g = pl.program_id(1)
    At = At_ref[...]
    preds = (p0_ref, p1_ref, p2_ref, p3_ref, p4_ref)
    for j in range(_G):
        x = preds[j][0, 0]
        soft = 1.0 / (1.0 + jnp.exp(-x))
        hard = (soft > 0.5).astype(jnp.float32)
        num = jnp.sum(soft * hard)
        den = jnp.sum(hard)
        mask_score = num / (den + 1e-6)
        score = ss_s[b * _N + g * _G + j] * mask_score

        wide = _split3_dot(soft, At)
        for q in range(_NQCH):
            aq = Ab_ref[q * _CH:(q + 1) * _CH, :]
            wq = wide[_WSTART[q]:_WSTART[q] + _KW, :]
            upq = _split3_dot_rhs(aq, wq)
            out_ref[0, j, q * _CH:(q + 1) * _CH, :] = jnp.where(
                upq > 0.5, tgt_ref[0, j, q * _CH:(q + 1) * _CH, :], 0.0)
        score_ref[0, j] = jnp.full((1, 128), score, dtype=jnp.float32)


def kernel(pred_logits, pred_masks, tgt_masks, tgt_labels):
    B, Q, C = pred_logits.shape
    N = tgt_masks.shape[1]
    labels3 = tgt_labels.astype(jnp.int32).reshape(B, 1, N)

    idx, ss = pl.pallas_call(
        _match_body,
        grid=(B,),
        in_specs=[
            pl.BlockSpec((1, 1, N), lambda b: (b, 0, 0)),
            pl.BlockSpec((1, Q, C), lambda b: (b, 0, 0)),
        ],
        out_specs=[
            pl.BlockSpec((1, 1, N), lambda b: (b, 0, 0)),
            pl.BlockSpec((1, 1, N), lambda b: (b, 0, 0)),
        ],
        out_shape=[
            jax.ShapeDtypeStruct((B, 1, N), jnp.int32),
            jax.ShapeDtypeStruct((B, 1, N), jnp.float32),
        ],
    )(labels3, pred_logits)

    idx_flat = idx.reshape(B * N)
    ss_flat = ss.reshape(B * N)
    At = jnp.asarray(_A_NP.T.copy()).astype(jnp.bfloat16)
    Ab = jnp.asarray(_ABAND_NP).astype(jnp.bfloat16)

    def pred_spec(j):
        return pl.BlockSpec(
            (1, 1, _h, _w),
            lambda b, g, idx_s, ss_s: (b, idx_s[b * N + g * _G + j], 0, 0))

    masks, scores_pad = pl.pallas_call(
        _dense_body,
        grid_spec=pltpu.PrefetchScalarGridSpec(
            num_scalar_prefetch=2,
            grid=(B, N // _G),
            in_specs=[
                pred_spec(0), pred_spec(1), pred_spec(2), pred_spec(3),
                pred_spec(4),
                pl.BlockSpec((1, _G, _H, _W),
                             lambda b, g, idx_s, ss_s: (b, g, 0, 0)),
                pl.BlockSpec((_h, _W), lambda b, g, idx_s, ss_s: (0, 0)),
                pl.BlockSpec((_H, _KW), lambda b, g, idx_s, ss_s: (0, 0)),
            ],
            out_specs=[
                pl.BlockSpec((1, _G, _H, _W),
                             lambda b, g, idx_s, ss_s: (b, g, 0, 0)),
                pl.BlockSpec((1, _G, 1, 128),
                             lambda b, g, idx_s, ss_s: (b, g, 0, 0)),
            ],
        ),
        out_shape=[
            jax.ShapeDtypeStruct((B, N, _H, _W), jnp.float32),
            jax.ShapeDtypeStruct((B, N, 1, 128), jnp.float32),
        ],
    )(idx_flat, ss_flat, pred_masks, pred_masks, pred_masks, pred_masks,
      pred_masks, tgt_masks, At, Ab)

    return scores_pad[:, :, 0, 0], masks


# speedup vs baseline: 1.4219x; 1.0889x over previous
import numpy as np

import jax
import jax.numpy as jnp
from jax import lax
from jax.experimental import pallas as pl
from jax.experimental.pallas import tpu as pltpu

_B, _Q, _C = 2, 100, 81
_N = 20
_h = _w = 128
_H = _W = 512
_NQCH = 4
_CH = _H // _NQCH
_KW = 64
_WSTART = tuple(min(max(32 * q - 8, 0), _h - _KW) for q in range(_NQCH))


def _interp_matrix(out_size: int, in_size: int) -> np.ndarray:
    o = np.arange(out_size, dtype=np.float32)
    src = (o + 0.5) * (in_size / out_size) - 0.5
    i0f = np.floor(src)
    frac = (src - i0f).astype(np.float32)
    i0 = np.clip(i0f.astype(np.int64), 0, in_size - 1)
    i1 = np.clip(i0f.astype(np.int64) + 1, 0, in_size - 1)
    A = np.zeros((out_size, in_size), dtype=np.float32)
    A[o.astype(np.int64), i0] += 1.0 - frac
    A[o.astype(np.int64), i1] += frac
    return A


_A_NP = _interp_matrix(_H, _h)
_ABAND_NP = np.stack(
    [_A_NP[q * _CH:(q + 1) * _CH, _WSTART[q]:_WSTART[q] + _KW]
     for q in range(_NQCH)], axis=0)
_ABAND_NP = _ABAND_NP.reshape(_NQCH * _CH, _KW)


def _match_body(labels_ref, logits_ref, idx_ref, ss_ref):
    logits = logits_ref[0]
    mx = jnp.max(logits, axis=-1, keepdims=True)
    e = jnp.exp(logits - mx)
    prob = e / jnp.sum(e, axis=-1, keepdims=True)
    labels = labels_ref[0]
    iota_c = lax.broadcasted_iota(jnp.int32, (_C, _N), 0)
    onehot = (iota_c == labels).astype(jnp.float32)
    probT = lax.dot_general(onehot, prob, (((0,), (1,)), ((), ())),
                            precision=lax.Precision.HIGHEST,
                            preferred_element_type=jnp.float32)

    iota_row = lax.broadcasted_iota(jnp.int32, (_N, _Q), 0)
    iota_lane = lax.broadcasted_iota(jnp.int32, (1, _Q), 1)
    iota_tn = lax.broadcasted_iota(jnp.int32, (1, _N), 1)

    def step(t, carry):
        used, idxv, ssv = carry
        row = jnp.sum(jnp.where(iota_row == t, probT, 0.0), axis=0,
                      keepdims=True)
        c = jnp.where(used > 0.5, -jnp.inf, row)
        m = jnp.max(c)
        j = jnp.min(jnp.where(c == m, iota_lane, _Q))
        sel = iota_tn == t
        idxv = jnp.where(sel, j, idxv)
        ssv = jnp.where(sel, m, ssv)
        used = jnp.where(iota_lane == j, 1.0, used)
        return used, idxv, ssv

    used0 = jnp.zeros((1, _Q), dtype=jnp.float32)
    _, idxv, ssv = lax.fori_loop(
        0, _N, step,
        (used0, jnp.zeros((1, _N), jnp.int32), jnp.zeros((1, _N), jnp.float32)))
    idx_ref[0] = idxv
    ss_ref[0] = ssv


def _split3(x):
    hi = x.astype(jnp.bfloat16)
    r1 = x - hi.astype(jnp.float32)
    mid = r1.astype(jnp.bfloat16)
    lo = (r1 - mid.astype(jnp.float32)).astype(jnp.bfloat16)
    return hi, mid, lo


def _split3_dot(x, a3_bf16):
    hi, mid, lo = _split3(x)
    x3 = jnp.concatenate([hi, mid, lo], axis=1)
    return jnp.dot(x3, a3_bf16, preferred_element_type=jnp.float32)


def _split3_dot_rhs(a3_bf16, x):
    hi, mid, lo = _split3(x)
    x3 = jnp.concatenate([hi, mid, lo], axis=0)
    return jnp.dot(a3_bf16, x3, preferred_element_type=jnp.float32)


_G = 5


def _dense_body(idx_s, ss_s, p0_ref, p1_ref, p2_ref, p3_ref, p4_ref,
                tgt_ref, At_ref, Ab_ref, out_ref, score_ref):
    b = pl.program_id(0)
    g = pl.program_id(1)
    At = At_ref[...]
    preds = (p0_ref, p1_ref, p2_ref, p3_ref, p4_ref)
    for j in range(_G):
        x = preds[j][0, 0]
        soft = 1.0 / (1.0 + jnp.exp(-x))
        hard = (soft > 0.5).astype(jnp.float32)
        num = jnp.sum(soft * hard)
        den = jnp.sum(hard)
        mask_score = num / (den + 1e-6)
        score = ss_s[b * _N + g * _G + j] * mask_score

        wide = _split3_dot(soft, At)
        for q in range(_NQCH):
            aq = Ab_ref[q * _CH:(q + 1) * _CH, :]
            wq = wide[_WSTART[q]:_WSTART[q] + _KW, :]
            upq = _split3_dot_rhs(aq, wq)
            out_ref[0, j, q * _CH:(q + 1) * _CH, :] = jnp.where(
                upq > 0.5, tgt_ref[0, j, q * _CH:(q + 1) * _CH, :], 0.0)
        score_ref[0, j] = jnp.full((1, 128), score, dtype=jnp.float32)


def kernel(pred_logits, pred_masks, tgt_masks, tgt_labels):
    B, Q, C = pred_logits.shape
    N = tgt_masks.shape[1]
    labels3 = tgt_labels.astype(jnp.int32).reshape(B, 1, N)

    idx, ss = pl.pallas_call(
        _match_body,
        grid=(B,),
        in_specs=[
            pl.BlockSpec((1, 1, N), lambda b: (b, 0, 0)),
            pl.BlockSpec((1, Q, C), lambda b: (b, 0, 0)),
        ],
        out_specs=[
            pl.BlockSpec((1, 1, N), lambda b: (b, 0, 0)),
            pl.BlockSpec((1, 1, N), lambda b: (b, 0, 0)),
        ],
        out_shape=[
            jax.ShapeDtypeStruct((B, 1, N), jnp.int32),
            jax.ShapeDtypeStruct((B, 1, N), jnp.float32),
        ],
    )(labels3, pred_logits)

    idx_flat = idx.reshape(B * N)
    ss_flat = ss.reshape(B * N)
    At = jnp.asarray(np.concatenate([_A_NP.T] * 3, axis=0).copy()).astype(jnp.bfloat16)
    Ab = jnp.asarray(np.concatenate([_ABAND_NP] * 3, axis=1)).astype(jnp.bfloat16)

    def pred_spec(j):
        return pl.BlockSpec(
            (1, 1, _h, _w),
            lambda b, g, idx_s, ss_s: (b, idx_s[b * N + g * _G + j], 0, 0))

    masks, scores_pad = pl.pallas_call(
        _dense_body,
        grid_spec=pltpu.PrefetchScalarGridSpec(
            num_scalar_prefetch=2,
            grid=(B, N // _G),
            in_specs=[
                pred_spec(0), pred_spec(1), pred_spec(2), pred_spec(3),
                pred_spec(4),
                pl.BlockSpec((1, _G, _H, _W),
                             lambda b, g, idx_s, ss_s: (b, g, 0, 0)),
                pl.BlockSpec((3 * _h, _W), lambda b, g, idx_s, ss_s: (0, 0)),
                pl.BlockSpec((_H, 3 * _KW), lambda b, g, idx_s, ss_s: (0, 0)),
            ],
            out_specs=[
                pl.BlockSpec((1, _G, _H, _W),
                             lambda b, g, idx_s, ss_s: (b, g, 0, 0)),
                pl.BlockSpec((1, _G, 1, 128),
                             lambda b, g, idx_s, ss_s: (b, g, 0, 0)),
            ],
        ),
        out_shape=[
            jax.ShapeDtypeStruct((B, N, _H, _W), jnp.float32),
            jax.ShapeDtypeStruct((B, N, 1, 128), jnp.float32),
        ],
    )(idx_flat, ss_flat, pred_masks, pred_masks, pred_masks, pred_masks,
      pred_masks, tgt_masks, At, Ab)

    return scores_pad[:, :, 0, 0], masks
